# initial kernel scaffold (unmeasured)
import jax
import jax.numpy as jnp
from jax import lax
from jax.experimental import pallas as pl
from jax.experimental.pallas import tpu as pltpu

N_DEV = 16


def kernel(x, w_mat):
    m, k_local = x.shape
    _, n = w_mat.shape
    chunk = m // N_DEV

    def body(x_ref, w_ref, out_ref, send_buf, recv_buf, send_sems, recv_sems):
        my = lax.axis_index("i")
        left = lax.rem(my + N_DEV - 1, N_DEV)
        right = lax.rem(my + 1, N_DEV)

        barrier_sem = pltpu.get_barrier_semaphore()
        for nbr in (left, right):
            pl.semaphore_signal(
                barrier_sem, inc=1,
                device_id=(nbr,), device_id_type=pl.DeviceIdType.MESH,
            )
        pl.semaphore_wait(barrier_sem, 2)

        xb = x_ref[...].astype(jnp.bfloat16)
        wb = w_ref[...].astype(jnp.bfloat16)

        def partial_chunk(c):
            xs = lax.dynamic_slice_in_dim(xb, c * chunk, chunk, axis=0)
            return jnp.dot(xs, wb, preferred_element_type=jnp.float32)

        for s in range(N_DEV - 1):
            c_send = lax.rem(my + 2 * N_DEV - 1 - s, N_DEV)
            acc = partial_chunk(c_send)
            if s > 0:
                acc = acc + recv_buf[s - 1].astype(jnp.float32)
            send_buf[s] = acc.astype(jnp.bfloat16)
            rdma = pltpu.make_async_remote_copy(
                src_ref=send_buf.at[s],
                dst_ref=recv_buf.at[s],
                send_sem=send_sems.at[s],
                recv_sem=recv_sems.at[s],
                device_id=(right,),
                device_id_type=pl.DeviceIdType.MESH,
            )
            rdma.start()
            rdma.wait()

        total = partial_chunk(my) + recv_buf[N_DEV - 2].astype(jnp.float32)
        out_ref[...] = jnp.maximum(total, 0.0)

    return pl.pallas_call(
        body,
        out_shape=jax.ShapeDtypeStruct((chunk, n), jnp.float32),
        in_specs=[
            pl.BlockSpec(memory_space=pltpu.VMEM),
            pl.BlockSpec(memory_space=pltpu.VMEM),
        ],
        out_specs=pl.BlockSpec(memory_space=pltpu.VMEM),
        scratch_shapes=[
            pltpu.VMEM((N_DEV - 1, chunk, n), jnp.bfloat16),
            pltpu.VMEM((N_DEV - 1, chunk, n), jnp.bfloat16),
            pltpu.SemaphoreType.DMA((N_DEV - 1,)),
            pltpu.SemaphoreType.DMA((N_DEV - 1,)),
        ],
        compiler_params=pltpu.CompilerParams(collective_id=0),
    )(x, w_mat)


# baseline (device time: 56483 ns/iter reference)
import jax
import jax.numpy as jnp
from jax import lax
from jax.experimental import pallas as pl
from jax.experimental.pallas import tpu as pltpu

N_DEV = 16


def kernel(x, w_mat):
    m, k_local = x.shape
    _, n = w_mat.shape
    chunk = m // N_DEV

    def body(x_ref, w_ref, out_ref, send_buf, recv_buf, send_sems, recv_sems):
        my = lax.axis_index("i")
        left = lax.rem(my + N_DEV - 1, N_DEV)
        right = lax.rem(my + 1, N_DEV)

        barrier_sem = pltpu.get_barrier_semaphore()
        for nbr in (left, right):
            pl.semaphore_signal(
                barrier_sem, inc=1,
                device_id=(nbr,), device_id_type=pl.DeviceIdType.MESH,
            )
        pl.semaphore_wait(barrier_sem, 2)

        wb = w_ref[...].astype(jnp.bfloat16)

        def partial_chunk(c):
            xs = x_ref[pl.ds(c * chunk, chunk), :].astype(jnp.bfloat16)
            return jnp.dot(xs, wb, preferred_element_type=jnp.float32)

        for s in range(N_DEV - 1):
            c_send = lax.rem(my + 2 * N_DEV - 1 - s, N_DEV)
            acc = partial_chunk(c_send)
            if s > 0:
                acc = acc + recv_buf[s - 1].astype(jnp.float32)
            send_buf[s] = acc.astype(jnp.bfloat16)
            rdma = pltpu.make_async_remote_copy(
                src_ref=send_buf.at[s],
                dst_ref=recv_buf.at[s],
                send_sem=send_sems.at[s],
                recv_sem=recv_sems.at[s],
                device_id=(right,),
                device_id_type=pl.DeviceIdType.MESH,
            )
            rdma.start()
            rdma.wait()

        total = partial_chunk(my) + recv_buf[N_DEV - 2].astype(jnp.float32)
        out_ref[...] = jnp.maximum(total, 0.0)

    return pl.pallas_call(
        body,
        out_shape=jax.ShapeDtypeStruct((chunk, n), jnp.float32),
        in_specs=[
            pl.BlockSpec(memory_space=pltpu.VMEM),
            pl.BlockSpec(memory_space=pltpu.VMEM),
        ],
        out_specs=pl.BlockSpec(memory_space=pltpu.VMEM),
        scratch_shapes=[
            pltpu.VMEM((N_DEV - 1, chunk, n), jnp.bfloat16),
            pltpu.VMEM((N_DEV - 1, chunk, n), jnp.bfloat16),
            pltpu.SemaphoreType.DMA((N_DEV - 1,)),
            pltpu.SemaphoreType.DMA((N_DEV - 1,)),
        ],
        compiler_params=pltpu.CompilerParams(collective_id=0),
    )(x, w_mat)


# device time: 38810 ns/iter; 1.4554x vs baseline; 1.4554x over previous
import jax
import jax.numpy as jnp
from jax import lax
from jax.experimental import pallas as pl
from jax.experimental.pallas import tpu as pltpu

N_DEV = 16
NP = 4


def kernel(x, w_mat):
    m, k_local = x.shape
    _, n = w_mat.shape
    chunk = m // N_DEV
    group_rows = NP * chunk

    def body(x_ref, w_ref, out_ref, a_ref,
             p1_send, p1_recv, p2_send, p2_recv,
             p1_ssem, p1_rsem, p2_ssem, p2_rsem):
        my = lax.axis_index("i")
        z = my // NP
        j = lax.rem(my, NP)
        plane_left = z * NP + lax.rem(j + NP - 1, NP)
        plane_right = z * NP + lax.rem(j + 1, NP)
        z_left = lax.rem(z + NP - 1, NP) * NP + j
        z_right = lax.rem(z + 1, NP) * NP + j

        barrier_sem = pltpu.get_barrier_semaphore()
        for nbr in (plane_left, plane_right, z_left, z_right):
            pl.semaphore_signal(
                barrier_sem, inc=1,
                device_id=(nbr,), device_id_type=pl.DeviceIdType.MESH,
            )
        pl.semaphore_wait(barrier_sem, 4)

        wb = w_ref[...].astype(jnp.bfloat16)

        def group_partial(g):
            xs = jnp.concatenate(
                [x_ref[pl.ds((NP * zz + g) * chunk, chunk), :] for zz in range(NP)]
            ).astype(jnp.bfloat16)
            return jnp.dot(xs, wb, preferred_element_type=jnp.float32)

        for s in range(NP - 1):
            g_send = lax.rem(j + 2 * NP - 1 - s, NP)
            acc = group_partial(g_send)
            if s > 0:
                acc = acc + p1_recv[s - 1].astype(jnp.float32)
            p1_send[s] = acc.astype(jnp.bfloat16)
            rdma = pltpu.make_async_remote_copy(
                src_ref=p1_send.at[s],
                dst_ref=p1_recv.at[s],
                send_sem=p1_ssem.at[s],
                recv_sem=p1_rsem.at[s],
                device_id=(plane_right,),
                device_id_type=pl.DeviceIdType.MESH,
            )
            rdma.start()
            rdma.wait()

        a_ref[...] = group_partial(j) + p1_recv[NP - 2].astype(jnp.float32)

        for t in range(NP - 1):
            b_send = lax.rem(z + 2 * NP - 1 - t, NP)
            acc = a_ref[pl.ds(b_send * chunk, chunk), :]
            if t > 0:
                acc = acc + p2_recv[t - 1].astype(jnp.float32)
            p2_send[t] = acc.astype(jnp.bfloat16)
            rdma = pltpu.make_async_remote_copy(
                src_ref=p2_send.at[t],
                dst_ref=p2_recv.at[t],
                send_sem=p2_ssem.at[t],
                recv_sem=p2_rsem.at[t],
                device_id=(z_right,),
                device_id_type=pl.DeviceIdType.MESH,
            )
            rdma.start()
            rdma.wait()

        total = a_ref[pl.ds(z * chunk, chunk), :] + p2_recv[NP - 2].astype(jnp.float32)
        out_ref[...] = jnp.maximum(total, 0.0)

    return pl.pallas_call(
        body,
        out_shape=jax.ShapeDtypeStruct((chunk, n), jnp.float32),
        in_specs=[
            pl.BlockSpec(memory_space=pltpu.VMEM),
            pl.BlockSpec(memory_space=pltpu.VMEM),
        ],
        out_specs=pl.BlockSpec(memory_space=pltpu.VMEM),
        scratch_shapes=[
            pltpu.VMEM((group_rows, n), jnp.float32),
            pltpu.VMEM((NP - 1, group_rows, n), jnp.bfloat16),
            pltpu.VMEM((NP - 1, group_rows, n), jnp.bfloat16),
            pltpu.VMEM((NP - 1, chunk, n), jnp.bfloat16),
            pltpu.VMEM((NP - 1, chunk, n), jnp.bfloat16),
            pltpu.SemaphoreType.DMA((NP - 1,)),
            pltpu.SemaphoreType.DMA((NP - 1,)),
            pltpu.SemaphoreType.DMA((NP - 1,)),
            pltpu.SemaphoreType.DMA((NP - 1,)),
        ],
        compiler_params=pltpu.CompilerParams(collective_id=0),
    )(x, w_mat)


# device time: 28766 ns/iter; 1.9635x vs baseline; 1.3492x over previous
import jax
import jax.numpy as jnp
from jax import lax
from jax.experimental import pallas as pl
from jax.experimental.pallas import tpu as pltpu

N_DEV = 16
NP = 4


def kernel(x, w_mat):
    m, k_local = x.shape
    _, n = w_mat.shape
    chunk = m // N_DEV
    group_rows = NP * chunk
    hc = n // 2

    def body(x_ref, w_ref, out_ref, p_ref, a_ref,
             cw_send, cw_recv, ccw_send, ccw_recv, p2_send, p2_recv,
             cw_ssem, cw_rsem, ccw_ssem, ccw_rsem, p2_ssem, p2_rsem):
        my = lax.axis_index("i")
        z = my // NP
        j = lax.rem(my, NP)
        plane_left = z * NP + lax.rem(j + NP - 1, NP)
        plane_right = z * NP + lax.rem(j + 1, NP)

        barrier_sem = pltpu.get_barrier_semaphore()
        z_peers = [lax.rem(z + dz, NP) * NP + j for dz in (1, 2, 3)]
        for nbr in [plane_left, plane_right] + z_peers:
            pl.semaphore_signal(
                barrier_sem, inc=1,
                device_id=(nbr,), device_id_type=pl.DeviceIdType.MESH,
            )
        pl.semaphore_wait(barrier_sem, 5)

        xp = jnp.concatenate(
            [
                x_ref[(NP * zz + g) * chunk:(NP * zz + g + 1) * chunk, :]
                for g in range(NP)
                for zz in range(NP)
            ]
        ).astype(jnp.bfloat16)
        wb = w_ref[...].astype(jnp.bfloat16)
        p_ref[...] = jnp.dot(xp, wb, preferred_element_type=jnp.float32)

        def pgroup(g, lo, width):
            return p_ref[pl.ds(g * group_rows, group_rows), lo:lo + width]

        for s in range(NP - 1):
            g_cw = lax.rem(j + 2 * NP - 1 - s, NP)
            g_ccw = lax.rem(j + 1 + s, NP)
            acc_cw = pgroup(g_cw, 0, hc)
            acc_ccw = pgroup(g_ccw, hc, hc)
            if s > 0:
                acc_cw = acc_cw + cw_recv[s - 1].astype(jnp.float32)
                acc_ccw = acc_ccw + ccw_recv[s - 1].astype(jnp.float32)
            cw_send[s] = acc_cw.astype(jnp.bfloat16)
            ccw_send[s] = acc_ccw.astype(jnp.bfloat16)
            rdma_cw = pltpu.make_async_remote_copy(
                src_ref=cw_send.at[s],
                dst_ref=cw_recv.at[s],
                send_sem=cw_ssem.at[s],
                recv_sem=cw_rsem.at[s],
                device_id=(plane_right,),
                device_id_type=pl.DeviceIdType.MESH,
            )
            rdma_ccw = pltpu.make_async_remote_copy(
                src_ref=ccw_send.at[s],
                dst_ref=ccw_recv.at[s],
                send_sem=ccw_ssem.at[s],
                recv_sem=ccw_rsem.at[s],
                device_id=(plane_left,),
                device_id_type=pl.DeviceIdType.MESH,
            )
            rdma_cw.start()
            rdma_ccw.start()
            rdma_cw.wait()
            rdma_ccw.wait()

        a_ref[:, 0:hc] = pgroup(j, 0, hc) + cw_recv[NP - 2].astype(jnp.float32)
        a_ref[:, hc:n] = pgroup(j, hc, hc) + ccw_recv[NP - 2].astype(jnp.float32)

        p2_rdmas = []
        for r in (1, 2, 3):
            b = lax.rem(z + NP - r, NP)
            p2_send[r - 1] = a_ref[pl.ds(b * chunk, chunk), :].astype(jnp.bfloat16)
            rdma = pltpu.make_async_remote_copy(
                src_ref=p2_send.at[r - 1],
                dst_ref=p2_recv.at[r - 1],
                send_sem=p2_ssem.at[r - 1],
                recv_sem=p2_rsem.at[r - 1],
                device_id=(lax.rem(z + NP - r, NP) * NP + j,),
                device_id_type=pl.DeviceIdType.MESH,
            )
            rdma.start()
            p2_rdmas.append(rdma)

        for rdma in p2_rdmas:
            rdma.wait_recv()
        total = (
            a_ref[pl.ds(z * chunk, chunk), :]
            + p2_recv[0].astype(jnp.float32)
            + p2_recv[1].astype(jnp.float32)
            + p2_recv[2].astype(jnp.float32)
        )
        out_ref[...] = jnp.maximum(total, 0.0)
        for rdma in p2_rdmas:
            rdma.wait_send()

    return pl.pallas_call(
        body,
        out_shape=jax.ShapeDtypeStruct((chunk, n), jnp.float32),
        in_specs=[
            pl.BlockSpec(memory_space=pltpu.VMEM),
            pl.BlockSpec(memory_space=pltpu.VMEM),
        ],
        out_specs=pl.BlockSpec(memory_space=pltpu.VMEM),
        scratch_shapes=[
            pltpu.VMEM((m, n), jnp.float32),
            pltpu.VMEM((group_rows, n), jnp.float32),
            pltpu.VMEM((NP - 1, group_rows, hc), jnp.bfloat16),
            pltpu.VMEM((NP - 1, group_rows, hc), jnp.bfloat16),
            pltpu.VMEM((NP - 1, group_rows, hc), jnp.bfloat16),
            pltpu.VMEM((NP - 1, group_rows, hc), jnp.bfloat16),
            pltpu.VMEM((NP - 1, chunk, n), jnp.bfloat16),
            pltpu.VMEM((NP - 1, chunk, n), jnp.bfloat16),
            pltpu.SemaphoreType.DMA((NP - 1,)),
            pltpu.SemaphoreType.DMA((NP - 1,)),
            pltpu.SemaphoreType.DMA((NP - 1,)),
            pltpu.SemaphoreType.DMA((NP - 1,)),
            pltpu.SemaphoreType.DMA((NP - 1,)),
            pltpu.SemaphoreType.DMA((NP - 1,)),
        ],
        compiler_params=pltpu.CompilerParams(collective_id=0),
    )(x, w_mat)


# device time: 26400 ns/iter; 2.1395x vs baseline; 1.0896x over previous
import jax
import jax.numpy as jnp
from jax import lax
from jax.experimental import pallas as pl
from jax.experimental.pallas import tpu as pltpu

N_DEV = 16
NP = 4
NS = 2


def kernel(x, w_mat):
    m, k_local = x.shape
    _, n = w_mat.shape
    chunk = m // N_DEV
    group_rows = NP * chunk
    hc = n // 2
    qc = hc // NS

    def body(x_ref, w_ref, out_ref, p_ref, a_ref,
             cw_send, cw_recv, ccw_send, ccw_recv, p2_send, p2_recv,
             cw_ssem, cw_rsem, ccw_ssem, ccw_rsem, p2_ssem, p2_rsem):
        my = lax.axis_index("i")
        z = my // NP
        j = lax.rem(my, NP)
        plane_left = z * NP + lax.rem(j + NP - 1, NP)
        plane_right = z * NP + lax.rem(j + 1, NP)

        barrier_sem = pltpu.get_barrier_semaphore()
        z_peers = [lax.rem(z + dz, NP) * NP + j for dz in (1, 2, 3)]
        for nbr in [plane_left, plane_right] + z_peers:
            pl.semaphore_signal(
                barrier_sem, inc=1,
                device_id=(nbr,), device_id_type=pl.DeviceIdType.MESH,
            )

        xp = jnp.concatenate(
            [
                x_ref[(NP * zz + g) * chunk:(NP * zz + g + 1) * chunk, :]
                for g in range(NP)
                for zz in range(NP)
            ]
        ).astype(jnp.bfloat16)
        wb = w_ref[...].astype(jnp.bfloat16)
        p_ref[...] = jnp.dot(xp, wb, preferred_element_type=jnp.float32)

        pl.semaphore_wait(barrier_sem, 5)

        def pgroup(g, lo, width):
            return p_ref[pl.ds(g * group_rows, group_rows), lo:lo + width]

        streams = [
            (cw_send, cw_recv, cw_ssem, cw_rsem, k * qc, plane_right, +1)
            for k in range(NS)
        ] + [
            (ccw_send, ccw_recv, ccw_ssem, ccw_rsem, hc + k * qc, plane_left, -1)
            for k in range(NS)
        ]
        live = {}
        for s in range(NP - 1):
            g_cw = lax.rem(j + 2 * NP - 1 - s, NP)
            g_ccw = lax.rem(j + 1 + s, NP)
            for idx, (sbuf, rbuf, ssem, rsem, lo, target, sgn) in enumerate(streams):
                k = idx % NS
                g = g_cw if sgn > 0 else g_ccw
                acc = pgroup(g, lo, qc)
                if s > 0:
                    live[(idx, s - 1)].wait_recv()
                    acc = acc + rbuf[s - 1, k].astype(jnp.float32)
                sbuf[s, k] = acc.astype(jnp.bfloat16)
                rdma = pltpu.make_async_remote_copy(
                    src_ref=sbuf.at[s, k],
                    dst_ref=rbuf.at[s, k],
                    send_sem=ssem.at[s, k],
                    recv_sem=rsem.at[s, k],
                    device_id=(target,),
                    device_id_type=pl.DeviceIdType.MESH,
                )
                rdma.start()
                live[(idx, s)] = rdma

        for idx, (sbuf, rbuf, ssem, rsem, lo, target, sgn) in enumerate(streams):
            k = idx % NS
            live[(idx, NP - 2)].wait_recv()
            a_ref[:, lo:lo + qc] = (
                pgroup(j, lo, qc) + rbuf[NP - 2, k].astype(jnp.float32)
            )

        p2_rdmas = []
        for r in (1, 2, 3):
            b = lax.rem(z + NP - r, NP)
            p2_send[r - 1] = a_ref[pl.ds(b * chunk, chunk), :].astype(jnp.bfloat16)
            rdma = pltpu.make_async_remote_copy(
                src_ref=p2_send.at[r - 1],
                dst_ref=p2_recv.at[r - 1],
                send_sem=p2_ssem.at[r - 1],
                recv_sem=p2_rsem.at[r - 1],
                device_id=(b * NP + j,),
                device_id_type=pl.DeviceIdType.MESH,
            )
            rdma.start()
            p2_rdmas.append(rdma)

        for rdma in p2_rdmas:
            rdma.wait_recv()
        total = (
            a_ref[pl.ds(z * chunk, chunk), :]
            + p2_recv[0].astype(jnp.float32)
            + p2_recv[1].astype(jnp.float32)
            + p2_recv[2].astype(jnp.float32)
        )
        out_ref[...] = jnp.maximum(total, 0.0)

        for rdma in live.values():
            rdma.wait_send()
        for rdma in p2_rdmas:
            rdma.wait_send()

    return pl.pallas_call(
        body,
        out_shape=jax.ShapeDtypeStruct((chunk, n), jnp.float32),
        in_specs=[
            pl.BlockSpec(memory_space=pltpu.VMEM),
            pl.BlockSpec(memory_space=pltpu.VMEM),
        ],
        out_specs=pl.BlockSpec(memory_space=pltpu.VMEM),
        scratch_shapes=[
            pltpu.VMEM((m, n), jnp.float32),
            pltpu.VMEM((group_rows, n), jnp.float32),
            pltpu.VMEM((NP - 1, NS, group_rows, qc), jnp.bfloat16),
            pltpu.VMEM((NP - 1, NS, group_rows, qc), jnp.bfloat16),
            pltpu.VMEM((NP - 1, NS, group_rows, qc), jnp.bfloat16),
            pltpu.VMEM((NP - 1, NS, group_rows, qc), jnp.bfloat16),
            pltpu.VMEM((NP - 1, chunk, n), jnp.bfloat16),
            pltpu.VMEM((NP - 1, chunk, n), jnp.bfloat16),
            pltpu.SemaphoreType.DMA((NP - 1, NS)),
            pltpu.SemaphoreType.DMA((NP - 1, NS)),
            pltpu.SemaphoreType.DMA((NP - 1, NS)),
            pltpu.SemaphoreType.DMA((NP - 1, NS)),
            pltpu.SemaphoreType.DMA((NP - 1,)),
            pltpu.SemaphoreType.DMA((NP - 1,)),
        ],
        compiler_params=pltpu.CompilerParams(collective_id=0),
    )(x, w_mat)


# device time: 25841 ns/iter; 2.1858x vs baseline; 1.0216x over previous
import jax
import jax.numpy as jnp
from jax import lax
from jax.experimental import pallas as pl
from jax.experimental.pallas import tpu as pltpu

N_DEV = 16
NP = 4
NS = 4


def kernel(x, w_mat):
    m, k_local = x.shape
    _, n = w_mat.shape
    chunk = m // N_DEV
    group_rows = NP * chunk
    hc = n // 2
    qc = hc // NS

    def body(x_ref, w_ref, out_ref, p_ref,
             cw_send, cw_recv, ccw_send, ccw_recv, p2_send, p2_recv,
             cw_ssem, cw_rsem, ccw_ssem, ccw_rsem, p2_ssem, p2_rsem):
        my = lax.axis_index("i")
        z = my // NP
        j = lax.rem(my, NP)
        plane_left = z * NP + lax.rem(j + NP - 1, NP)
        plane_right = z * NP + lax.rem(j + 1, NP)

        barrier_sem = pltpu.get_barrier_semaphore()
        z_peers = [lax.rem(z + dz, NP) * NP + j for dz in (1, 2, 3)]
        for nbr in [plane_left, plane_right] + z_peers:
            pl.semaphore_signal(
                barrier_sem, inc=1,
                device_id=(nbr,), device_id_type=pl.DeviceIdType.MESH,
            )

        xp = jnp.concatenate(
            [
                x_ref[(NP * zz + g) * chunk:(NP * zz + g + 1) * chunk, :]
                for g in range(NP)
                for zz in range(NP)
            ]
        ).astype(jnp.bfloat16)
        wb = w_ref[...].astype(jnp.bfloat16)
        p_ref[...] = jnp.dot(xp, wb, preferred_element_type=jnp.float32)

        pl.semaphore_wait(barrier_sem, 5)

        def pgroup(g, lo, width):
            return p_ref[pl.ds(g * group_rows, group_rows), lo:lo + width]

        streams = [
            (cw_send, cw_recv, cw_ssem, cw_rsem, k * qc, plane_right, +1)
            for k in range(NS)
        ] + [
            (ccw_send, ccw_recv, ccw_ssem, ccw_rsem, hc + k * qc, plane_left, -1)
            for k in range(NS)
        ]
        live = {}
        for s in range(NP - 1):
            g_cw = lax.rem(j + 2 * NP - 1 - s, NP)
            g_ccw = lax.rem(j + 1 + s, NP)
            for idx, (sbuf, rbuf, ssem, rsem, lo, target, sgn) in enumerate(streams):
                k = idx % NS
                g = g_cw if sgn > 0 else g_ccw
                acc = pgroup(g, lo, qc)
                if s > 0:
                    live[(idx, s - 1)].wait_recv()
                    acc = acc + rbuf[s - 1, k].astype(jnp.float32)
                sbuf[s, k] = acc.astype(jnp.bfloat16)
                rdma = pltpu.make_async_remote_copy(
                    src_ref=sbuf.at[s, k],
                    dst_ref=rbuf.at[s, k],
                    send_sem=ssem.at[s, k],
                    recv_sem=rsem.at[s, k],
                    device_id=(target,),
                    device_id_type=pl.DeviceIdType.MESH,
                )
                rdma.start()
                live[(idx, s)] = rdma

        for idx in range(len(streams)):
            live[(idx, NP - 2)].wait_recv()

        def reduced_block(b, lo_k):
            sbuf, rbuf, ssem, rsem, lo, target, sgn = streams[lo_k]
            k = lo_k % NS
            return (
                p_ref[pl.ds(j * group_rows + b * chunk, chunk), lo:lo + qc]
                + rbuf[NP - 2, k, pl.ds(b * chunk, chunk), :].astype(jnp.float32)
            )

        p2_rdmas = []
        for r in (1, 2, 3):
            b = lax.rem(z + NP - r, NP)
            p2_send[r - 1] = jnp.concatenate(
                [reduced_block(b, lo_k) for lo_k in range(2 * NS)], axis=1
            ).astype(jnp.bfloat16)
            rdma = pltpu.make_async_remote_copy(
                src_ref=p2_send.at[r - 1],
                dst_ref=p2_recv.at[r - 1],
                send_sem=p2_ssem.at[r - 1],
                recv_sem=p2_rsem.at[r - 1],
                device_id=(b * NP + j,),
                device_id_type=pl.DeviceIdType.MESH,
            )
            rdma.start()
            p2_rdmas.append(rdma)

        own = jnp.concatenate(
            [reduced_block(z, lo_k) for lo_k in range(2 * NS)], axis=1
        )
        for rdma in p2_rdmas:
            rdma.wait_recv()
        total = (
            own
            + p2_recv[0].astype(jnp.float32)
            + p2_recv[1].astype(jnp.float32)
            + p2_recv[2].astype(jnp.float32)
        )
        out_ref[...] = jnp.maximum(total, 0.0)

        for rdma in live.values():
            rdma.wait_send()
        for rdma in p2_rdmas:
            rdma.wait_send()

    return pl.pallas_call(
        body,
        out_shape=jax.ShapeDtypeStruct((chunk, n), jnp.float32),
        in_specs=[
            pl.BlockSpec(memory_space=pltpu.VMEM),
            pl.BlockSpec(memory_space=pltpu.VMEM),
        ],
        out_specs=pl.BlockSpec(memory_space=pltpu.VMEM),
        scratch_shapes=[
            pltpu.VMEM((m, n), jnp.float32),
            pltpu.VMEM((NP - 1, NS, group_rows, qc), jnp.bfloat16),
            pltpu.VMEM((NP - 1, NS, group_rows, qc), jnp.bfloat16),
            pltpu.VMEM((NP - 1, NS, group_rows, qc), jnp.bfloat16),
            pltpu.VMEM((NP - 1, NS, group_rows, qc), jnp.bfloat16),
            pltpu.VMEM((NP - 1, chunk, n), jnp.bfloat16),
            pltpu.VMEM((NP - 1, chunk, n), jnp.bfloat16),
            pltpu.SemaphoreType.DMA((NP - 1, NS)),
            pltpu.SemaphoreType.DMA((NP - 1, NS)),
            pltpu.SemaphoreType.DMA((NP - 1, NS)),
            pltpu.SemaphoreType.DMA((NP - 1, NS)),
            pltpu.SemaphoreType.DMA((NP - 1,)),
            pltpu.SemaphoreType.DMA((NP - 1,)),
        ],
        compiler_params=pltpu.CompilerParams(collective_id=0),
    )(x, w_mat)


# device time: 23442 ns/iter; 2.4095x vs baseline; 1.1023x over previous
import contextlib
import os

import jax
import jax.numpy as jnp
from jax import lax
from jax.experimental import pallas as pl
from jax.experimental.pallas import tpu as pltpu


def _scope(name):
    if os.environ.get("KERNEL_SCOPES") == "1":
        return jax.named_scope(name)
    return contextlib.nullcontext()


N_DEV = 16
NP = 4
NS = 4


def kernel(x, w_mat):
    m, k_local = x.shape
    _, n = w_mat.shape
    chunk = m // N_DEV
    group_rows = NP * chunk
    hc = n // 2
    qc = hc // NS

    def body(x_ref, w_ref, out_ref, p_ref,
             cw_send, cw_recv, ccw_send, ccw_recv, p2_send, p2_recv,
             cw_ssem, cw_rsem, ccw_ssem, ccw_rsem, p2_ssem, p2_rsem):
        my = lax.axis_index("i")
        z = my // NP
        j = lax.rem(my, NP)
        plane_left = z * NP + lax.rem(j + NP - 1, NP)
        plane_right = z * NP + lax.rem(j + 1, NP)

        barrier_sem = pltpu.get_barrier_semaphore()
        for nbr in [plane_left, plane_right]:
            pl.semaphore_signal(
                barrier_sem, inc=1,
                device_id=(nbr,), device_id_type=pl.DeviceIdType.MESH,
            )

        with _scope("gemm"):
            xp = jnp.concatenate(
                [
                    x_ref[(NP * zz + g) * chunk:(NP * zz + g + 1) * chunk, :]
                    for g in range(NP)
                    for zz in range(NP)
                ]
            ).astype(jnp.bfloat16)
            wb = w_ref[...].astype(jnp.bfloat16)
            p_ref[...] = jnp.dot(xp, wb, preferred_element_type=jnp.float32)

        with _scope("barrier_wait"):
            pl.semaphore_wait(barrier_sem, 2)

        def pgroup(g, lo, width):
            return p_ref[pl.ds(g * group_rows, group_rows), lo:lo + width]

        ablate = os.environ.get("ABLATE", "")

        streams = [
            (cw_send, cw_recv, cw_ssem, cw_rsem, k * qc, plane_right, +1)
            for k in range(NS)
        ] + [
            (ccw_send, ccw_recv, ccw_ssem, ccw_rsem, hc + k * qc, plane_left, -1)
            for k in range(NS)
        ]
        live = {}
        for s in range(NP - 1) if ablate not in ("p2", "none") else []:
            g_cw = lax.rem(j + 2 * NP - 1 - s, NP)
            g_ccw = lax.rem(j + 1 + s, NP)
            with _scope(f"hop{s}"):
                for idx, (sbuf, rbuf, ssem, rsem, lo, target, sgn) in enumerate(streams):
                    k = idx % NS
                    g = g_cw if sgn > 0 else g_ccw
                    acc = pgroup(g, lo, qc)
                    if s > 0:
                        live[(idx, s - 1)].wait_recv()
                        acc = acc + rbuf[s - 1, k].astype(jnp.float32)
                    sbuf[s, k] = acc.astype(jnp.bfloat16)
                    rdma = pltpu.make_async_remote_copy(
                        src_ref=sbuf.at[s, k],
                        dst_ref=rbuf.at[s, k],
                        send_sem=ssem.at[s, k],
                        recv_sem=rsem.at[s, k],
                        device_id=(target,),
                        device_id_type=pl.DeviceIdType.MESH,
                    )
                    rdma.start()
                    live[(idx, s)] = rdma

        def reduced_block(b, lo_k):
            sbuf, rbuf, ssem, rsem, lo, target, sgn = streams[lo_k]
            k = lo_k % NS
            base = p_ref[pl.ds(j * group_rows + b * chunk, chunk), lo:lo + qc]
            if ablate in ("p2", "none"):
                return base
            return (
                base
                + rbuf[NP - 2, k, pl.ds(b * chunk, chunk), :].astype(jnp.float32)
            )

        p2_rdmas = []
        own_pieces = []
        with _scope("p2_send"):
            for lo_k in range(2 * NS):
                if ablate not in ("p2", "none"):
                    live[(lo_k, NP - 2)].wait_recv()
                for r in (1, 2, 3) if ablate not in ("p1", "none") else []:
                    b = lax.rem(z + NP - r, NP)
                    p2_send[r - 1, lo_k] = reduced_block(b, lo_k).astype(
                        jnp.bfloat16
                    )
                    rdma = pltpu.make_async_remote_copy(
                        src_ref=p2_send.at[r - 1, lo_k],
                        dst_ref=p2_recv.at[r - 1, lo_k],
                        send_sem=p2_ssem.at[r - 1, lo_k],
                        recv_sem=p2_rsem.at[r - 1, lo_k],
                        device_id=(b * NP + j,),
                        device_id_type=pl.DeviceIdType.MESH,
                    )
                    rdma.start()
                    p2_rdmas.append(rdma)
                own_pieces.append(reduced_block(z, lo_k))

        with _scope("p2_recv"):
            for rdma in p2_rdmas:
                rdma.wait_recv()
        with _scope("out"):
            total = jnp.concatenate(own_pieces, axis=1)
            if ablate not in ("p1", "none"):
                for r in (1, 2, 3):
                    total = total + jnp.concatenate(
                        [p2_recv[r - 1, lo_k] for lo_k in range(2 * NS)], axis=1
                    ).astype(jnp.float32)
            out_ref[...] = jnp.maximum(total, 0.0)

            for rdma in live.values():
                rdma.wait_send()
            for rdma in p2_rdmas:
                rdma.wait_send()

    return pl.pallas_call(
        body,
        out_shape=jax.ShapeDtypeStruct((chunk, n), jnp.float32),
        in_specs=[
            pl.BlockSpec(memory_space=pltpu.VMEM),
            pl.BlockSpec(memory_space=pltpu.VMEM),
        ],
        out_specs=pl.BlockSpec(memory_space=pltpu.VMEM),
        scratch_shapes=[
            pltpu.VMEM((m, n), jnp.float32),
            pltpu.VMEM((NP - 1, NS, group_rows, qc), jnp.bfloat16),
            pltpu.VMEM((NP - 1, NS, group_rows, qc), jnp.bfloat16),
            pltpu.VMEM((NP - 1, NS, group_rows, qc), jnp.bfloat16),
            pltpu.VMEM((NP - 1, NS, group_rows, qc), jnp.bfloat16),
            pltpu.VMEM((NP - 1, 2 * NS, chunk, qc), jnp.bfloat16),
            pltpu.VMEM((NP - 1, 2 * NS, chunk, qc), jnp.bfloat16),
            pltpu.SemaphoreType.DMA((NP - 1, NS)),
            pltpu.SemaphoreType.DMA((NP - 1, NS)),
            pltpu.SemaphoreType.DMA((NP - 1, NS)),
            pltpu.SemaphoreType.DMA((NP - 1, NS)),
            pltpu.SemaphoreType.DMA((NP - 1, 2 * NS)),
            pltpu.SemaphoreType.DMA((NP - 1, 2 * NS)),
        ],
        compiler_params=pltpu.CompilerParams(collective_id=0),
    )(x, w_mat)
